# Initial kernel scaffold; baseline (speedup 1.0000x reference)
import jax
import jax.numpy as jnp
from jax.experimental import pallas as pl


def kernel(logits, regress, anchors):
    raise NotImplementedError("write your pallas kernel here")


# speedup vs baseline: 7.3377x; 7.3377x over previous
import math

import jax
import jax.numpy as jnp
from jax.experimental import pallas as pl
from jax.experimental.pallas import tpu as pltpu

_PROPOSALS = 100
_IOU_T = 0.3
_SCORE_T = 0.7
_MAXR = abs(math.log(16.0 / 1000.0))
_NEG = float("-inf")
_BIG = 2**30


def _decode(px1, py1, px2, py2, d0, d1, d2, d3):
    dx = d0 * 0.1
    dy = d1 * 0.1
    dw = jnp.clip(d2 * 0.2, -_MAXR, _MAXR)
    dh = jnp.clip(d3 * 0.2, -_MAXR, _MAXR)
    pw = px2 - px1
    ph = py2 - py1
    pcx = px1 + 0.5 * pw
    pcy = py1 + 0.5 * ph
    gw = pw * jnp.exp(dw)
    gh = ph * jnp.exp(dh)
    gcx = pcx + pw * dx
    gcy = pcy + ph * dy
    x1 = jnp.clip(gcx - 0.5 * gw, 0.0, 1.0)
    y1 = jnp.clip(gcy - 0.5 * gh, 0.0, 1.0)
    x2 = jnp.clip(gcx + 0.5 * gw, 0.0, 1.0)
    y2 = jnp.clip(gcy + 0.5 * gh, 0.0, 1.0)
    return x1, y1, x2, y2


def _body(n, npad, nclass, lT_ref, lN_ref, ancT_ref, regT_ref, ancN_ref,
          regN_ref, out_logit_ref, out_prop_ref, s_scr):
    nc = nclass - 1
    lT = lT_ref[...]
    l0 = lT[0:1]
    rest = lT[1:nclass]
    maxrest = jnp.max(rest, axis=0, keepdims=True)
    fg = maxrest > l0
    col = jax.lax.broadcasted_iota(jnp.int32, (1, npad), 1)
    inb = col < n

    ancT = ancT_ref[...]
    regT = regT_ref[...]
    bx1, by1, bx2, by2 = _decode(
        ancT[0:1], ancT[1:2], ancT[2:3], ancT[3:4],
        regT[0:1], regT[1:2], regT[2:3], regT[3:4])
    a2 = jnp.maximum(bx2 - bx1, 0.0) * jnp.maximum(by2 - by1, 0.0)

    valid = fg & inb & (rest >= _SCORE_T)
    s_scr[...] = jnp.where(valid, rest, _NEG)

    stepcol = jax.lax.broadcasted_iota(jnp.int32, (nc, _PROPOSALS), 1)

    def nms_step(t, carry):
        rec_i, rec_s = carry
        s = s_scr[...]
        maxv = jnp.max(s, axis=1, keepdims=True)
        ok = maxv > -1e37
        m1 = s == maxv
        idx = jnp.min(jnp.where(m1, col, _BIG), axis=1, keepdims=True)
        selm = col == idx
        px1 = jnp.sum(jnp.where(selm, bx1, 0.0), axis=1, keepdims=True)
        py1 = jnp.sum(jnp.where(selm, by1, 0.0), axis=1, keepdims=True)
        px2 = jnp.sum(jnp.where(selm, bx2, 0.0), axis=1, keepdims=True)
        py2 = jnp.sum(jnp.where(selm, by2, 0.0), axis=1, keepdims=True)
        a1 = jnp.maximum(px2 - px1, 0.0) * jnp.maximum(py2 - py1, 0.0)
        ix1 = jnp.maximum(bx1, px1)
        iy1 = jnp.maximum(by1, py1)
        ix2 = jnp.minimum(bx2, px2)
        iy2 = jnp.minimum(by2, py2)
        inter = jnp.maximum(ix2 - ix1, 0.0) * jnp.maximum(iy2 - iy1, 0.0)
        union = jnp.maximum(a1 + a2 - inter, 1e-8)
        kill = ((inter > _IOU_T * union) | selm) & ok
        s_scr[...] = jnp.where(kill, _NEG, s)
        recm = stepcol == t
        rec_i = jnp.where(recm, idx, rec_i)
        rec_s = jnp.where(recm, maxv, rec_s)
        return rec_i, rec_s

    rec_i, rec_s = jax.lax.fori_loop(
        0, _PROPOSALS, nms_step,
        (jnp.zeros((nc, _PROPOSALS), jnp.int32),
         jnp.full((nc, _PROPOSALS), _NEG, jnp.float32)))

    flati = (jax.lax.broadcasted_iota(jnp.int32, (nc, _PROPOSALS), 0)
             * _PROPOSALS + stepcol)

    def out_step(j, sc):
        maxv = jnp.max(sc)
        flat = jnp.min(jnp.where(sc == maxv, flati, _BIG))
        a = jnp.sum(jnp.where(flati == flat, rec_i, 0))
        okf = (maxv > -1e37).astype(jnp.float32)
        lrow = lN_ref[pl.ds(a, 1)]
        out_logit_ref[pl.ds(j, 1)] = lrow * okf
        anc = ancN_ref[pl.ds(a, 1)]
        reg = regN_ref[pl.ds(a, 1)]
        b = _decode(anc[..., 0:1], anc[..., 1:2], anc[..., 2:3],
                    anc[..., 3:4], reg[..., 0:1], reg[..., 1:2],
                    reg[..., 2:3], reg[..., 3:4])
        out_prop_ref[pl.ds(j, 1)] = jnp.concatenate(b, axis=-1) * okf
        return jnp.where(flati == flat, _NEG, sc)

    jax.lax.fori_loop(0, _PROPOSALS, out_step, rec_s)


def kernel(logits, regress, anchors):
    B, N, C = logits.shape
    npad = ((N + 127) // 128) * 128
    l = logits.reshape(N, C)
    r = regress.reshape(N, 4)
    lT = jnp.pad(l.T, ((0, 0), (0, npad - N)))
    regT = jnp.pad(r.T, ((0, 0), (0, npad - N)))
    ancT = jnp.pad(anchors.T, ((0, 0), (0, npad - N)))
    lN = l[:, None, :]
    ancN = anchors[:, None, :]
    regN = r[:, None, :]

    import functools
    body = functools.partial(_body, N, npad, C)
    out_logit, out_prop = pl.pallas_call(
        body,
        out_shape=[
            jax.ShapeDtypeStruct((_PROPOSALS, 1, C), jnp.float32),
            jax.ShapeDtypeStruct((_PROPOSALS, 1, 4), jnp.float32),
        ],
        scratch_shapes=[pltpu.VMEM((C - 1, npad), jnp.float32)],
    )(lT, lN, ancT, regT, ancN, regN)
    return (out_logit.reshape(B, _PROPOSALS, C),
            out_prop.reshape(B, _PROPOSALS, 4))


# speedup vs baseline: 7.7753x; 1.0596x over previous
import math

import jax
import jax.numpy as jnp
from jax.experimental import pallas as pl
from jax.experimental.pallas import tpu as pltpu

_PROPOSALS = 100
_IOU_T = 0.3
_SCORE_T = 0.7
_MAXR = abs(math.log(16.0 / 1000.0))
_NEG = float("-inf")
_BIG = 2**30


def _decode(px1, py1, px2, py2, d0, d1, d2, d3):
    dx = d0 * 0.1
    dy = d1 * 0.1
    dw = jnp.clip(d2 * 0.2, -_MAXR, _MAXR)
    dh = jnp.clip(d3 * 0.2, -_MAXR, _MAXR)
    pw = px2 - px1
    ph = py2 - py1
    pcx = px1 + 0.5 * pw
    pcy = py1 + 0.5 * ph
    gw = pw * jnp.exp(dw)
    gh = ph * jnp.exp(dh)
    gcx = pcx + pw * dx
    gcy = pcy + ph * dy
    x1 = jnp.clip(gcx - 0.5 * gw, 0.0, 1.0)
    y1 = jnp.clip(gcy - 0.5 * gh, 0.0, 1.0)
    x2 = jnp.clip(gcx + 0.5 * gw, 0.0, 1.0)
    y2 = jnp.clip(gcy + 0.5 * gh, 0.0, 1.0)
    return x1, y1, x2, y2


def _body(n, npad, nclass, lT_ref, lN_ref, ancT_ref, regT_ref, ancN_ref,
          regN_ref, out_logit_ref, out_prop_ref, s_scr):
    nc = nclass - 1
    lT = lT_ref[...]
    l0 = lT[0:1]
    rest = lT[1:nclass]
    maxrest = jnp.max(rest, axis=0, keepdims=True)
    fg = maxrest > l0
    col = jax.lax.broadcasted_iota(jnp.int32, (1, npad), 1)
    inb = col < n

    ancT = ancT_ref[...]
    regT = regT_ref[...]
    bx1, by1, bx2, by2 = _decode(
        ancT[0:1], ancT[1:2], ancT[2:3], ancT[3:4],
        regT[0:1], regT[1:2], regT[2:3], regT[3:4])
    a2 = jnp.maximum(bx2 - bx1, 0.0) * jnp.maximum(by2 - by1, 0.0)

    valid = fg & inb & (rest >= _SCORE_T)
    s_scr[...] = jnp.where(valid, rest, _NEG)

    stepcol = jax.lax.broadcasted_iota(jnp.int32, (nc, _PROPOSALS), 1)
    rowi = jax.lax.broadcasted_iota(jnp.int32, (nc, 1), 0)

    def nms_step(t, carry):
        rec_i, rec_s = carry
        s = s_scr[...]
        maxv = jnp.max(s, axis=1, keepdims=True)
        m1 = s == maxv
        idx = jnp.min(jnp.where(m1, col, _BIG), axis=1, keepdims=True)
        selm = col == idx
        ancs, regs = [], []
        for c in range(nc):
            a_c = jnp.sum(jnp.where(rowi == c, idx, 0))
            ancs.append(ancN_ref[pl.ds(a_c, 1)])
            regs.append(regN_ref[pl.ds(a_c, 1)])
        anc = jnp.concatenate(ancs, axis=0).reshape(nc, 4)
        reg = jnp.concatenate(regs, axis=0).reshape(nc, 4)
        px1, py1, px2, py2 = _decode(
            anc[:, 0:1], anc[:, 1:2], anc[:, 2:3], anc[:, 3:4],
            reg[:, 0:1], reg[:, 1:2], reg[:, 2:3], reg[:, 3:4])
        a1 = jnp.maximum(px2 - px1, 0.0) * jnp.maximum(py2 - py1, 0.0)
        ix1 = jnp.maximum(bx1, px1)
        iy1 = jnp.maximum(by1, py1)
        ix2 = jnp.minimum(bx2, px2)
        iy2 = jnp.minimum(by2, py2)
        inter = jnp.maximum(ix2 - ix1, 0.0) * jnp.maximum(iy2 - iy1, 0.0)
        union = jnp.maximum(a1 + a2 - inter, 1e-8)
        kill = (inter > _IOU_T * union) | selm
        s_scr[...] = jnp.where(kill, _NEG, s)
        recm = stepcol == t
        rec_i = jnp.where(recm, idx, rec_i)
        rec_s = jnp.where(recm, maxv, rec_s)
        return rec_i, rec_s

    rec_i, rec_s = jax.lax.fori_loop(
        0, _PROPOSALS, nms_step,
        (jnp.zeros((nc, _PROPOSALS), jnp.int32),
         jnp.full((nc, _PROPOSALS), _NEG, jnp.float32)))

    flati = (jax.lax.broadcasted_iota(jnp.int32, (nc, _PROPOSALS), 0)
             * _PROPOSALS + stepcol)

    def out_step(j, sc):
        maxv = jnp.max(sc)
        flat = jnp.min(jnp.where(sc == maxv, flati, _BIG))
        a = jnp.sum(jnp.where(flati == flat, rec_i, 0))
        okf = (maxv > -1e37).astype(jnp.float32)
        lrow = lN_ref[pl.ds(a, 1)]
        out_logit_ref[pl.ds(j, 1)] = lrow * okf
        anc = ancN_ref[pl.ds(a, 1)]
        reg = regN_ref[pl.ds(a, 1)]
        b = _decode(anc[..., 0:1], anc[..., 1:2], anc[..., 2:3],
                    anc[..., 3:4], reg[..., 0:1], reg[..., 1:2],
                    reg[..., 2:3], reg[..., 3:4])
        out_prop_ref[pl.ds(j, 1)] = jnp.concatenate(b, axis=-1) * okf
        return jnp.where(flati == flat, _NEG, sc)

    jax.lax.fori_loop(0, _PROPOSALS, out_step, rec_s)


def kernel(logits, regress, anchors):
    B, N, C = logits.shape
    npad = ((N + 127) // 128) * 128
    l = logits.reshape(N, C)
    r = regress.reshape(N, 4)
    lT = jnp.pad(l.T, ((0, 0), (0, npad - N)))
    regT = jnp.pad(r.T, ((0, 0), (0, npad - N)))
    ancT = jnp.pad(anchors.T, ((0, 0), (0, npad - N)))
    lN = l[:, None, :]
    ancN = anchors[:, None, :]
    regN = r[:, None, :]

    import functools
    body = functools.partial(_body, N, npad, C)
    out_logit, out_prop = pl.pallas_call(
        body,
        out_shape=[
            jax.ShapeDtypeStruct((_PROPOSALS, 1, C), jnp.float32),
            jax.ShapeDtypeStruct((_PROPOSALS, 1, 4), jnp.float32),
        ],
        scratch_shapes=[pltpu.VMEM((C - 1, npad), jnp.float32)],
    )(lT, lN, ancT, regT, ancN, regN)
    return (out_logit.reshape(B, _PROPOSALS, C),
            out_prop.reshape(B, _PROPOSALS, 4))


# speedup vs baseline: 12.7211x; 1.6361x over previous
import math

import jax
import jax.numpy as jnp
from jax.experimental import pallas as pl
from jax.experimental.pallas import tpu as pltpu

_PROPOSALS = 100
_IOU_T = 0.3
_SCORE_T = 0.7
_MAXR = abs(math.log(16.0 / 1000.0))
_NEG = float("-inf")
_BIG = 2**30
_SUB = 8


def _decode(px1, py1, px2, py2, d0, d1, d2, d3):
    dx = d0 * 0.1
    dy = d1 * 0.1
    dw = jnp.clip(d2 * 0.2, -_MAXR, _MAXR)
    dh = jnp.clip(d3 * 0.2, -_MAXR, _MAXR)
    pw = px2 - px1
    ph = py2 - py1
    pcx = px1 + 0.5 * pw
    pcy = py1 + 0.5 * ph
    gw = pw * jnp.exp(dw)
    gh = ph * jnp.exp(dh)
    gcx = pcx + pw * dx
    gcy = pcy + ph * dy
    x1 = jnp.clip(gcx - 0.5 * gw, 0.0, 1.0)
    y1 = jnp.clip(gcy - 0.5 * gh, 0.0, 1.0)
    x2 = jnp.clip(gcx + 0.5 * gw, 0.0, 1.0)
    y2 = jnp.clip(gcy + 0.5 * gh, 0.0, 1.0)
    return x1, y1, x2, y2


def _body(n, lpad, nclass, lT_ref, lN_ref, ancT_ref, regT_ref, ancN_ref,
          regN_ref, out_logit_ref, out_prop_ref, s_scr):
    nc = nclass - 1
    lT = lT_ref[...]
    l0 = lT[0:1]
    rest = lT[1:nclass]
    maxrest = jnp.max(rest, axis=0, keepdims=True)
    fg = maxrest > l0
    iota_s = jax.lax.broadcasted_iota(jnp.int32, (1, _SUB, lpad), 1)
    iota_l = jax.lax.broadcasted_iota(jnp.int32, (1, _SUB, lpad), 2)
    col = iota_s * lpad + iota_l
    inb = col < n

    ancT = ancT_ref[...]
    regT = regT_ref[...]
    bx1, by1, bx2, by2 = _decode(
        ancT[0:1], ancT[1:2], ancT[2:3], ancT[3:4],
        regT[0:1], regT[1:2], regT[2:3], regT[3:4])
    a2 = jnp.maximum(bx2 - bx1, 0.0) * jnp.maximum(by2 - by1, 0.0)

    valid = fg & inb & (rest >= _SCORE_T)
    s_scr[...] = jnp.where(valid, rest, _NEG)

    stepcol = jax.lax.broadcasted_iota(jnp.int32, (nc, _PROPOSALS), 1)
    rowi = jax.lax.broadcasted_iota(jnp.int32, (nc, 1), 0)

    def nms_step(t, carry):
        rec_i, rec_s = carry
        s = s_scr[...]
        maxv = jnp.max(jnp.max(s, axis=2, keepdims=True), axis=1,
                       keepdims=True)
        m1 = s == maxv
        cand = jnp.where(m1, col, _BIG)
        idx = jnp.min(jnp.min(cand, axis=2, keepdims=True), axis=1,
                      keepdims=True)
        selm = col == idx
        idx2 = idx.reshape(nc, 1)
        ancs, regs = [], []
        for c in range(nc):
            a_c = jnp.sum(jnp.where(rowi == c, idx2, 0))
            ancs.append(ancN_ref[pl.ds(a_c, 1)])
            regs.append(regN_ref[pl.ds(a_c, 1)])
        anc = jnp.concatenate(ancs, axis=0).reshape(nc, 4)
        reg = jnp.concatenate(regs, axis=0).reshape(nc, 4)
        px1, py1, px2, py2 = _decode(
            anc[:, 0:1], anc[:, 1:2], anc[:, 2:3], anc[:, 3:4],
            reg[:, 0:1], reg[:, 1:2], reg[:, 2:3], reg[:, 3:4])
        a1 = (jnp.maximum(px2 - px1, 0.0)
              * jnp.maximum(py2 - py1, 0.0)).reshape(nc, 1, 1)
        px1 = px1.reshape(nc, 1, 1)
        py1 = py1.reshape(nc, 1, 1)
        px2 = px2.reshape(nc, 1, 1)
        py2 = py2.reshape(nc, 1, 1)
        ix1 = jnp.maximum(bx1, px1)
        iy1 = jnp.maximum(by1, py1)
        ix2 = jnp.minimum(bx2, px2)
        iy2 = jnp.minimum(by2, py2)
        inter = jnp.maximum(ix2 - ix1, 0.0) * jnp.maximum(iy2 - iy1, 0.0)
        union = jnp.maximum(a1 + a2 - inter, 1e-8)
        kill = (inter > _IOU_T * union) | selm
        s_scr[...] = jnp.where(kill, _NEG, s)
        recm = stepcol == t
        rec_i = jnp.where(recm, idx2, rec_i)
        rec_s = jnp.where(recm, maxv.reshape(nc, 1), rec_s)
        return rec_i, rec_s

    rec_i, rec_s = jax.lax.fori_loop(
        0, _PROPOSALS, nms_step,
        (jnp.zeros((nc, _PROPOSALS), jnp.int32),
         jnp.full((nc, _PROPOSALS), _NEG, jnp.float32)))

    flati = (jax.lax.broadcasted_iota(jnp.int32, (nc, _PROPOSALS), 0)
             * _PROPOSALS + stepcol)

    def out_step(j, sc):
        maxv = jnp.max(sc)
        flat = jnp.min(jnp.where(sc == maxv, flati, _BIG))
        a = jnp.sum(jnp.where(flati == flat, rec_i, 0))
        okf = (maxv > -1e37).astype(jnp.float32)
        lrow = lN_ref[pl.ds(a, 1)]
        out_logit_ref[pl.ds(j, 1)] = lrow * okf
        anc = ancN_ref[pl.ds(a, 1)]
        reg = regN_ref[pl.ds(a, 1)]
        b = _decode(anc[..., 0:1], anc[..., 1:2], anc[..., 2:3],
                    anc[..., 3:4], reg[..., 0:1], reg[..., 1:2],
                    reg[..., 2:3], reg[..., 3:4])
        out_prop_ref[pl.ds(j, 1)] = jnp.concatenate(b, axis=-1) * okf
        return jnp.where(flati == flat, _NEG, sc)

    jax.lax.fori_loop(0, _PROPOSALS, out_step, rec_s)


def kernel(logits, regress, anchors):
    B, N, C = logits.shape
    lpad = ((N + (_SUB * 128) - 1) // (_SUB * 128)) * 128
    npad = _SUB * lpad
    l = logits.reshape(N, C)
    r = regress.reshape(N, 4)
    lT = jnp.pad(l.T, ((0, 0), (0, npad - N))).reshape(C, _SUB, lpad)
    regT = jnp.pad(r.T, ((0, 0), (0, npad - N))).reshape(4, _SUB, lpad)
    ancT = jnp.pad(anchors.T, ((0, 0), (0, npad - N))).reshape(4, _SUB, lpad)
    lN = l[:, None, :]
    ancN = anchors[:, None, :]
    regN = r[:, None, :]

    import functools
    body = functools.partial(_body, N, lpad, C)
    out_logit, out_prop = pl.pallas_call(
        body,
        out_shape=[
            jax.ShapeDtypeStruct((_PROPOSALS, 1, C), jnp.float32),
            jax.ShapeDtypeStruct((_PROPOSALS, 1, 4), jnp.float32),
        ],
        scratch_shapes=[pltpu.VMEM((C - 1, _SUB, lpad), jnp.float32)],
    )(lT, lN, ancT, regT, ancN, regN)
    return (out_logit.reshape(B, _PROPOSALS, C),
            out_prop.reshape(B, _PROPOSALS, 4))
